# baseline (speedup 1.0000x reference)
import functools

import jax
import jax.numpy as jnp
from jax import lax
from jax.experimental import pallas as pl
from jax.experimental.pallas import tpu as pltpu
from jax.experimental.pallas import tpu_sc as plsc

_NC, _NS, _L = 2, 16, 16
_NW = _NC * _NS
_B, _S, _D = 4, 8192, 1024
_SB = _S // _NW
_C = 16
_NCH = _SB // _C
_CHW = _C * _D


def _sc_body(x_hbm, t_hbm, o_hbm, tbuf, xb0, xb1, xb2, xb3,
             ld0, ld1, ld2, ld3, st0, st1, st2, st3):
    cid = lax.axis_index("c")
    sid = lax.axis_index("s")
    wid = sid * _NC + cid
    s0 = wid * _SB

    xbs = (xb0, xb1, xb2, xb3)
    lds = (ld0, ld1, ld2, ld3)
    sts = (st0, st1, st2, st3)

    def chunk(ci, carry):
        srow = s0 + ci * _C
        handles = []
        for b in range(4):
            @pl.when(ci > 0)
            def _(b=b):
                pltpu.make_async_copy(
                    xbs[b], o_hbm.at[pl.ds(0, _CHW)], sts[b]).wait()
            xoff = (b * _S + srow) * _D
            handles.append(
                pltpu.async_copy(x_hbm.at[pl.ds(xoff, _CHW)], xbs[b], lds[b]))
        pltpu.sync_copy(t_hbm.at[pl.ds(srow * _D, _CHW)], tbuf)
        for b in range(4):
            handles[b].wait()
            buf = xbs[b]

            def add_body(k, c, buf=buf):
                sl = pl.ds(k * _L, _L)
                plsc.addupdate(buf.at[sl], tbuf[sl])
                return c

            lax.fori_loop(0, _CHW // _L, add_body, None, unroll=8)
            xoff = (b * _S + srow) * _D
            pltpu.async_copy(buf, o_hbm.at[pl.ds(xoff, _CHW)], sts[b])
        return carry

    lax.fori_loop(0, _NCH, chunk, None)
    for b in range(4):
        pltpu.make_async_copy(xbs[b], o_hbm.at[pl.ds(0, _CHW)], sts[b]).wait()


@jax.jit
def _sc_add(x_flat, t_flat):
    mesh = plsc.VectorSubcoreMesh(
        core_axis_name="c", subcore_axis_name="s",
        num_cores=_NC, num_subcores=_NS)
    f = pl.kernel(
        _sc_body,
        out_type=jax.ShapeDtypeStruct((_B * _S * _D,), jnp.float32),
        mesh=mesh,
        scratch_types=(
            [pltpu.VMEM((_CHW,), jnp.float32)] * 5
            + [pltpu.SemaphoreType.DMA] * 8
        ),
    )
    return f(x_flat, t_flat)


def kernel(input_embeddings, emb_table):
    B, S, D = input_embeddings.shape
    x = input_embeddings.reshape(B * S * D)
    t = emb_table.reshape(-1)
    return _sc_add(x, t).reshape(B, S, D)


# speedup vs baseline: 1.1682x; 1.1682x over previous
import functools

import jax
import jax.numpy as jnp
from jax import lax
from jax.experimental import pallas as pl
from jax.experimental.pallas import tpu as pltpu
from jax.experimental.pallas import tpu_sc as plsc

_NC, _NS, _L = 2, 16, 16
_NW = _NC * _NS
_B, _S, _D = 4, 8192, 1024
_SB = _S // _NW
_C = 16
_NCH = _SB // _C


def _sc_body(x_hbm, t_hbm, o_hbm, tbuf, xb0, xb1, xb2, xb3,
             ld0, ld1, ld2, ld3, st0, st1, st2, st3):
    cid = lax.axis_index("c")
    sid = lax.axis_index("s")
    wid = sid * _NC + cid
    s0 = wid * _SB

    xbs = (xb0, xb1, xb2, xb3)
    lds = (ld0, ld1, ld2, ld3)
    sts = (st0, st1, st2, st3)

    def chunk(ci, carry):
        srow = s0 + ci * _C
        rows = pl.ds(srow, _C)
        handles = []
        for b in range(4):
            @pl.when(ci > 0)
            def _(b=b):
                pltpu.make_async_copy(
                    xbs[b], o_hbm.at[b, pl.ds(0, _C), :], sts[b]).wait()
            handles.append(
                pltpu.async_copy(x_hbm.at[b, rows, :], xbs[b], lds[b]))
        pltpu.sync_copy(t_hbm.at[rows, :], tbuf)
        for b in range(4):
            handles[b].wait()
            buf = xbs[b]

            def row_body(r, c, buf=buf):
                def col_body(j, c2):
                    sl = pl.ds(j * _L, _L)
                    plsc.addupdate(buf.at[r, sl], tbuf[r, sl])
                    return c2
                return lax.fori_loop(0, _D // _L, col_body, c, unroll=8)

            lax.fori_loop(0, _C, row_body, None)
            pltpu.async_copy(buf, o_hbm.at[b, rows, :], sts[b])
        return carry

    lax.fori_loop(0, _NCH, chunk, None)
    for b in range(4):
        pltpu.make_async_copy(
            xbs[b], o_hbm.at[b, pl.ds(0, _C), :], sts[b]).wait()


@jax.jit
def _sc_add(x, t):
    mesh = plsc.VectorSubcoreMesh(
        core_axis_name="c", subcore_axis_name="s",
        num_cores=_NC, num_subcores=_NS)
    f = pl.kernel(
        _sc_body,
        out_type=jax.ShapeDtypeStruct((_B, _S, _D), jnp.float32),
        mesh=mesh,
        scratch_types=(
            [pltpu.VMEM((_C, _D), jnp.float32)] * 5
            + [pltpu.SemaphoreType.DMA] * 8
        ),
        compiler_params=pltpu.CompilerParams(use_tc_tiling_on_sc=True),
    )
    return f(x, t)


def kernel(input_embeddings, emb_table):
    return _sc_add(input_embeddings, emb_table)


# speedup vs baseline: 1.7580x; 1.5048x over previous
import functools

import jax
import jax.numpy as jnp
from jax import lax
from jax.experimental import pallas as pl
from jax.experimental.pallas import tpu as pltpu
from jax.experimental.pallas import tpu_sc as plsc

_NC, _NS, _L = 2, 16, 16
_NW = _NC * _NS
_B, _S, _D = 4, 8192, 1024
_SB = _S // _NW
_C = 16
_NCH = _SB // _C


def _sc_body(x_hbm, t_hbm, o_hbm, tbuf, xb0, xb1, xb2, xb3,
             ld0, ld1, ld2, ld3, st0, st1, st2, st3):
    cid = lax.axis_index("c")
    sid = lax.axis_index("s")
    wid = sid * _NC + cid
    s0 = wid * _SB

    xbs = (xb0, xb1, xb2, xb3)
    lds = (ld0, ld1, ld2, ld3)
    sts = (st0, st1, st2, st3)

    def chunk(ci, carry):
        srow = s0 + ci * _C
        rows = pl.ds(srow, _C)
        handles = []
        for b in range(4):
            @pl.when(ci > 0)
            def _(b=b):
                pltpu.make_async_copy(
                    xbs[b], o_hbm.at[b, pl.ds(0, _C), :], sts[b]).wait()
            handles.append(
                pltpu.async_copy(x_hbm.at[b, rows, :], xbs[b], lds[b]))
        pltpu.sync_copy(t_hbm.at[rows, :], tbuf)
        for b in range(4):
            handles[b].wait()
            buf = xbs[b]

            def tile_body(tc_i, c, buf=buf):
                col0 = tc_i * 128
                for r in range(_C):
                    for cc in range(128 // _L):
                        sl = pl.ds(col0 + cc * _L, _L)
                        plsc.addupdate(buf.at[r, sl], tbuf[r, sl])
                return c

            lax.fori_loop(0, _D // 128, tile_body, None)
            pltpu.async_copy(buf, o_hbm.at[b, rows, :], sts[b])
        return carry

    lax.fori_loop(0, _NCH, chunk, None)
    for b in range(4):
        pltpu.make_async_copy(
            xbs[b], o_hbm.at[b, pl.ds(0, _C), :], sts[b]).wait()


@jax.jit
def _sc_add(x, t):
    mesh = plsc.VectorSubcoreMesh(
        core_axis_name="c", subcore_axis_name="s",
        num_cores=_NC, num_subcores=_NS)
    f = pl.kernel(
        _sc_body,
        out_type=jax.ShapeDtypeStruct((_B, _S, _D), jnp.float32),
        mesh=mesh,
        scratch_types=(
            [pltpu.VMEM((_C, _D), jnp.float32)] * 5
            + [pltpu.SemaphoreType.DMA] * 8
        ),
        compiler_params=pltpu.CompilerParams(use_tc_tiling_on_sc=True),
    )
    return f(x, t)


def kernel(input_embeddings, emb_table):
    return _sc_add(input_embeddings, emb_table)


# speedup vs baseline: 2.1752x; 1.2373x over previous
import functools

import jax
import jax.numpy as jnp
from jax import lax
from jax.experimental import pallas as pl
from jax.experimental.pallas import tpu as pltpu
from jax.experimental.pallas import tpu_sc as plsc

_NC, _NS, _L = 2, 16, 16
_NW = _NC * _NS
_B, _S, _D = 4, 8192, 1024
_SB = _S // _NW
_C = 16
_NCH = _SB // _C


def _sc_body(x_hbm, t_hbm, o_hbm, tbuf, xb0, xb1, xb2, xb3,
             ld0, ld1, ld2, ld3, st0, st1, st2, st3):
    cid = lax.axis_index("c")
    sid = lax.axis_index("s")
    wid = sid * _NC + cid
    s0 = wid * _SB

    xbs = (xb0, xb1, xb2, xb3)
    lds = (ld0, ld1, ld2, ld3)
    sts = (st0, st1, st2, st3)

    def chunk(ci, carry):
        srow = s0 + ci * _C
        rows = pl.ds(srow, _C)
        handles = []
        for b in range(4):
            @pl.when(ci > 0)
            def _(b=b):
                pltpu.make_async_copy(
                    xbs[b], o_hbm.at[b, pl.ds(0, _C), :], sts[b]).wait()
            handles.append(
                pltpu.async_copy(x_hbm.at[b, rows, :], xbs[b], lds[b]))
        pltpu.sync_copy(t_hbm.at[rows, :], tbuf)
        for b in range(4):
            handles[b].wait()

        def tile_body(tc_i, c):
            col0 = tc_i * 128
            for r in range(_C):
                for cc in range(128 // _L):
                    sl = pl.ds(col0 + cc * _L, _L)
                    v = tbuf[r, sl]
                    for b in range(4):
                        plsc.addupdate(xbs[b].at[r, sl], v)
            return c

        lax.fori_loop(0, _D // 128, tile_body, None)
        for b in range(4):
            pltpu.async_copy(xbs[b], o_hbm.at[b, rows, :], sts[b])
        return carry

    lax.fori_loop(0, _NCH, chunk, None)
    for b in range(4):
        pltpu.make_async_copy(
            xbs[b], o_hbm.at[b, pl.ds(0, _C), :], sts[b]).wait()


@jax.jit
def _sc_add(x, t):
    mesh = plsc.VectorSubcoreMesh(
        core_axis_name="c", subcore_axis_name="s",
        num_cores=_NC, num_subcores=_NS)
    f = pl.kernel(
        _sc_body,
        out_type=jax.ShapeDtypeStruct((_B, _S, _D), jnp.float32),
        mesh=mesh,
        scratch_types=(
            [pltpu.VMEM((_C, _D), jnp.float32)] * 5
            + [pltpu.SemaphoreType.DMA] * 8
        ),
        compiler_params=pltpu.CompilerParams(use_tc_tiling_on_sc=True),
    )
    return f(x, t)


def kernel(input_embeddings, emb_table):
    return _sc_add(input_embeddings, emb_table)
